# baseline (speedup 1.0000x reference)
import functools

import jax
import jax.numpy as jnp
from jax import lax
from jax.experimental import pallas as pl
from jax.experimental.pallas import tpu as pltpu
from jax.experimental.pallas import tpu_sc as plsc

NUM_CORES = 2
NUM_SUBCORES = 16
NUM_WORKERS = NUM_CORES * NUM_SUBCORES
TOTAL_ROWS = 4096 * 50
ROWS_PER_WORKER = TOTAL_ROWS // NUM_WORKERS
CHUNKS = (128, 256, 512, 640, 640, 640, 640, 640, 640, 640, 640, 256, 128)
assert sum(CHUNKS) == ROWS_PER_WORKER
MAX_CHUNK = max(CHUNKS)
D = 64


@functools.partial(
    pl.kernel,
    out_type=jax.ShapeDtypeStruct((TOTAL_ROWS, D), jnp.float32),
    mesh=plsc.VectorSubcoreMesh(core_axis_name="c", subcore_axis_name="s"),
    compiler_params=pltpu.CompilerParams(use_tc_tiling_on_sc=False),
    scratch_types=[
        pltpu.VMEM((ROWS_PER_WORKER,), jnp.int32),
        pltpu.VMEM((MAX_CHUNK, D), jnp.float32),
        pltpu.VMEM((MAX_CHUNK, D), jnp.float32),
        pltpu.SemaphoreType.DMA,
        pltpu.SemaphoreType.DMA,
        pltpu.SemaphoreType.DMA,
        pltpu.SemaphoreType.DMA,
        pltpu.SemaphoreType.DMA,
    ],
)
def _lookup(x_hbm, table_hbm, out_hbm, idx_v, rows_a, rows_b,
            sem_ga, sem_gb, sem_wa, sem_wb, sem_i):
    wid = lax.axis_index("s") * NUM_CORES + lax.axis_index("c")
    base = wid * ROWS_PER_WORKER
    offs = [sum(CHUNKS[:i]) for i in range(len(CHUNKS))]

    pltpu.sync_copy(
        x_hbm.at[wid, pl.ds(0, CHUNKS[0])], idx_v.at[pl.ds(0, CHUNKS[0])]
    )
    rest = ROWS_PER_WORKER - CHUNKS[0]
    idx_rest = pltpu.async_copy(
        x_hbm.at[wid, pl.ds(CHUNKS[0], rest)],
        idx_v.at[pl.ds(CHUNKS[0], rest)],
        sem_i,
    )

    bufs = (rows_a, rows_b)
    gsems = (sem_ga, sem_gb)
    wsems = (sem_wa, sem_wb)

    def fire(c, p):
        return pltpu.async_copy(
            table_hbm.at[idx_v.at[pl.ds(offs[c], CHUNKS[c])]],
            bufs[p].at[pl.ds(0, CHUNKS[c])],
            gsems[p],
        )

    def detect(c):
        def min_body(v, acc):
            return jnp.minimum(acc, idx_v[pl.ds(offs[c] + v * 16, 16)])

        acc = lax.fori_loop(
            0, CHUNKS[c] // 16, min_body, jnp.full((16,), 1, jnp.int32)
        )
        mn = acc[0]
        for i in range(1, 16):
            mn = jnp.minimum(mn, acc[i])
        return mn

    def fix(c, buf):
        def fix_body(v, _):
            vals = idx_v[pl.ds(offs[c] + v * 16, 16)]
            zeros16 = jnp.zeros((16,), jnp.float32)
            for lane in range(16):
                @pl.when(vals[lane] == 0)
                def _():
                    r = v * 16 + lane
                    buf[r, pl.ds(0, 16)] = zeros16
                    buf[r, pl.ds(16, 16)] = zeros16
                    buf[r, pl.ds(32, 16)] = zeros16
                    buf[r, pl.ds(48, 16)] = zeros16

            return 0

        lax.fori_loop(0, CHUNKS[c] // 16, fix_body, 0)

    n = len(CHUNKS)
    gathers = {0: fire(0, 0)}
    idx_pending = idx_rest
    writebacks = {}
    for c in range(n):
        p = c % 2
        buf, wsem = bufs[p], wsems[p]
        if c + 1 < n:
            if c - 1 in writebacks:
                writebacks.pop(c - 1).wait()
            if idx_pending is not None:
                idx_pending.wait()
                idx_pending = None
            gathers[c + 1] = fire(c + 1, 1 - p)
        mn = detect(c)
        gathers.pop(c).wait()
        pl.when(mn == 0)(lambda c=c, buf=buf: fix(c, buf))
        writebacks[c] = pltpu.async_copy(
            buf.at[pl.ds(0, CHUNKS[c])],
            out_hbm.at[pl.ds(base + offs[c], CHUNKS[c])],
            wsem,
        )
    writebacks.pop(n - 2).wait()
    writebacks.pop(n - 1).wait()


def kernel(x, table):
    x_flat = x.astype(jnp.int32).reshape(NUM_WORKERS, ROWS_PER_WORKER)
    out = _lookup(x_flat, table)
    return out.reshape(4096, 50, D)
